# baseline (speedup 1.0000x reference)
import functools

import jax
import jax.numpy as jnp
from jax import lax
from jax.experimental import pallas as pl
from jax.experimental.pallas import tpu as pltpu
from jax.experimental.pallas import tpu_sc as plsc

T = 2048
D = 1024
F = 1024
E = 8
BLK = 256
NB = 24
SPAD = NB * BLK
TB = 128
NTB = T // TB

_EPS = 1e-5
_ALPHA = 1.702
_LIMIT = 7.0

NC = 2
NS = 16
NW = NC * NS
TPW = T // NW
RPW = SPAD // NW
RCH = 48


def _routing_body(x_ref, nw_ref, wr_ref, br_ref,
                  t_ref, meta_ref, cnt_ref, carry_ref):
    b = pl.program_id(0)

    @pl.when(b == 0)
    def _():
        carry_ref[...] = jnp.zeros((1, TB), jnp.float32)

    x = x_ref[...]
    t = x * lax.rsqrt(jnp.mean(x * x, axis=1, keepdims=True) + _EPS) * nw_ref[...]
    t_ref[...] = t

    logits = lax.dot_general(t, wr_ref[...], (((1,), (1,)), ((), ())),
                             preferred_element_type=jnp.float32) + br_ref[...]
    iota_e = lax.broadcasted_iota(jnp.int32, (TB, E), 1).astype(jnp.float32)
    m0 = jnp.max(logits, axis=1, keepdims=True)
    e0 = jnp.min(jnp.where(logits >= m0, iota_e, 1e9), axis=1, keepdims=True)
    masked = jnp.where(iota_e == e0, -1e30, logits)
    m1 = jnp.max(masked, axis=1, keepdims=True)
    e1 = jnp.min(jnp.where(masked >= m1, iota_e, 1e9), axis=1, keepdims=True)
    w0 = jax.nn.sigmoid(m0 - m1)

    lane = lax.broadcasted_iota(jnp.int32, (TB, TB), 1).astype(jnp.float32)
    oh0 = (lane == e0).astype(jnp.float32)
    oh1 = (lane == e1).astype(jnp.float32)
    m_mat = oh0 + oh1
    row_i = lax.broadcasted_iota(jnp.int32, (TB, TB), 0).astype(jnp.float32)
    tri = (lane < row_i).astype(jnp.float32)
    carry = carry_ref[...]
    rank = lax.dot_general(tri, m_mat, (((1,), (0,)), ((), ())),
                           preferred_element_type=jnp.float32) + carry
    p0 = jnp.sum(rank * oh0, axis=1, keepdims=True)
    p1 = jnp.sum(rank * oh1, axis=1, keepdims=True)
    new_carry = carry + jnp.sum(m_mat, axis=0, keepdims=True)
    carry_ref[...] = new_carry

    p0hi = jnp.floor(p0 * (1.0 / 128.0))
    p0lo = p0 - 128.0 * p0hi
    p1hi = jnp.floor(p1 * (1.0 / 128.0))
    p1lo = p1 - 128.0 * p1hi
    w0hi = jnp.floor(w0 * 256.0 + 0.5) * (1.0 / 256.0)
    w0lo = w0 - w0hi
    cols = jnp.concatenate(
        [e0, e1, p0lo, p1lo, p0hi, p1hi, w0hi, w0lo], axis=1)
    eye = (lane == row_i).astype(jnp.float32)
    meta = lax.dot_general(cols, eye, (((0,), (0,)), ((), ())),
                           preferred_element_type=jnp.float32)
    meta_ref[...] = meta.reshape(1, E, TB)

    nblk = jnp.floor((new_carry + (BLK - 1)) * (1.0 / BLK))
    lte = (row_i <= lane).astype(jnp.float32)
    end_row = lax.dot_general(nblk, lte, (((1,), (0,)), ((), ())),
                              preferred_element_type=jnp.float32)
    end_col = lax.dot_general(eye, end_row, (((1,), (1,)), ((), ())),
                              preferred_element_type=jnp.float32)
    ge = (lane >= end_col).astype(jnp.float32) * (row_i < E).astype(jnp.float32)
    ones_row = jnp.ones((1, TB), jnp.float32)
    bexp_row = lax.dot_general(ones_row, ge, (((1,), (0,)), ((), ())),
                               preferred_element_type=jnp.float32)
    bexp_row = jnp.minimum(bexp_row, float(E - 1))
    shr = (row_i == lane - 1.0).astype(jnp.float32)
    prev_row = lax.dot_general(bexp_row, shr, (((1,), (0,)), ((), ())),
                               preferred_element_type=jnp.float32)
    lr = lane[0:1, :]
    change_row = ((bexp_row != prev_row).astype(jnp.float32)
                  * (lr > 0.0).astype(jnp.float32))
    runidx_row = lax.dot_general(change_row, lte, (((1,), (0,)), ((), ())),
                                 preferred_element_type=jnp.float32)
    cnt_ref[...] = jnp.concatenate(
        [new_carry, bexp_row, runidx_row,
         jnp.zeros((E - 3, TB), jnp.float32)],
        axis=0).astype(jnp.int32)


def _routing(x, norm_w, Wr, br):
    out_shape = [
        jax.ShapeDtypeStruct((T, D), jnp.float32),
        jax.ShapeDtypeStruct((NTB, E, TB), jnp.float32),
        jax.ShapeDtypeStruct((E, TB), jnp.int32),
    ]
    return pl.pallas_call(
        _routing_body,
        grid=(NTB,),
        in_specs=[
            pl.BlockSpec((TB, D), lambda b: (b, 0)),
            pl.BlockSpec((1, D), lambda b: (0, 0)),
            pl.BlockSpec((E, D), lambda b: (0, 0)),
            pl.BlockSpec((1, E), lambda b: (0, 0)),
        ],
        out_specs=[
            pl.BlockSpec((TB, D), lambda b: (b, 0)),
            pl.BlockSpec((1, E, TB), lambda b: (b, 0, 0)),
            pl.BlockSpec((E, TB), lambda b: (0, 0)),
        ],
        out_shape=out_shape,
        scratch_shapes=[pltpu.VMEM((1, TB), jnp.float32)],
        compiler_params=pltpu.CompilerParams(
            dimension_semantics=("arbitrary",)),
    )(x, norm_w.reshape(1, D), Wr, br.reshape(1, E))


def _vgather(vec, idx):
    dnums = lax.GatherDimensionNumbers(
        offset_dims=(), collapsed_slice_dims=(0,), start_index_map=(0,))
    return lax.gather(vec, idx.reshape(16, 1), dnums, (1,),
                      mode=lax.GatherScatterMode.PROMISE_IN_BOUNDS)


def _lane_splat(vec, lane):
    return _vgather(vec, jnp.full((16,), lane, jnp.int32))


def _cumsum16(vec):
    iota16 = lax.broadcasted_iota(jnp.int32, (16,), 0)
    x = vec
    for k in (1, 2, 4, 8):
        sh = _vgather(x, jnp.maximum(iota16 - k, 0))
        x = x + jnp.where(iota16 >= k, sh, jnp.zeros_like(x))
    return x


def _sc_positions(cnt_v, mv, hh_f, pb0, pb1):
    iota16 = lax.broadcasted_iota(jnp.int32, (16,), 0)
    zero16 = jnp.zeros((16,), jnp.int32)
    c_i = jnp.where(iota16 < E, cnt_v[...], zero16)
    shift = BLK.bit_length() - 1
    padded = ((c_i + (BLK - 1)) >> shift) << shift
    incl = _cumsum16(padded)
    off = incl - padded
    offs = [_lane_splat(off, e) for e in range(E)]

    def _half(row, g):
        a = mv[0, row, pl.ds(16 * g, 16)]
        b = mv[0, row, pl.ds(TPW + 16 * g, 16)]
        return a + (b - a) * hh_f

    for g in range(TPW // 16):
        e0g = _half(0, g).astype(jnp.int32)
        e1g = _half(1, g).astype(jnp.int32)
        acc0 = (_half(2, g) + 128.0 * _half(4, g)).astype(jnp.int32)
        acc1 = (_half(3, g) + 128.0 * _half(5, g)).astype(jnp.int32)
        for e in range(E):
            acc0 = acc0 + jnp.where(e0g == e, offs[e], zero16)
            acc1 = acc1 + jnp.where(e1g == e, offs[e], zero16)
        pb0[pl.ds(16 * g, 16)] = acc0
        pb1[pl.ds(16 * g, 16)] = acc1


def _b2_body(t_hbm, meta, cntb, xs_hbm, cnt_v, mv, pb0, pb1, rows, sem):
    wid = lax.axis_index("s") * NC + lax.axis_index("c")
    base = wid * TPW
    rr = wid >> 1
    hh = wid & 1

    pltpu.sync_copy(cntb.at[0, pl.ds(0, 16)], cnt_v)
    pltpu.sync_copy(meta.at[pl.ds(rr, 1), pl.ds(0, E), pl.ds(0, TB)], mv)
    cptr = pltpu.async_copy(t_hbm.at[pl.ds(base, TPW)], rows, sem)
    hh_f = jnp.broadcast_to(hh.astype(jnp.float32), (16,))
    _sc_positions(cnt_v, mv, hh_f, pb0, pb1)
    cptr.wait()
    cp0 = pltpu.async_copy(rows, xs_hbm.at[pb0], sem)
    cp1 = pltpu.async_copy(rows, xs_hbm.at[pb1], sem)
    cp0.wait()
    cp1.wait()


def _b2(t, meta, cntb):
    mesh = plsc.VectorSubcoreMesh(core_axis_name="c", subcore_axis_name="s")
    fn = functools.partial(
        pl.kernel, mesh=mesh,
        out_type=jax.ShapeDtypeStruct((SPAD, D), jnp.float32),
        scratch_types=[
            pltpu.VMEM((16,), jnp.int32),
            pltpu.VMEM((1, E, TB), jnp.float32),
            pltpu.VMEM((TPW,), jnp.int32),
            pltpu.VMEM((TPW,), jnp.int32),
            pltpu.VMEM((TPW, D), jnp.float32),
            pltpu.SemaphoreType.DMA,
        ],
    )(_b2_body)
    return fn(t, meta, cntb)


def _gmm_body(bexp_ref, xs_ref, wg_hbm, bg_ref, wu_hbm, bu_ref,
              wd_hbm, bd_ref, ys_ref, wg2, wu2, wd2, sems):
    b = pl.program_id(0)
    run = bexp_ref[2, b]
    slot = lax.rem(run, 2)

    def _copies(e, s):
        return (pltpu.make_async_copy(wg_hbm.at[e], wg2.at[s], sems.at[s, 0]),
                pltpu.make_async_copy(wu_hbm.at[e], wu2.at[s], sems.at[s, 1]),
                pltpu.make_async_copy(wd_hbm.at[e], wd2.at[s], sems.at[s, 2]))

    @pl.when(b == 0)
    def _():
        for cp in _copies(bexp_ref[1, 0], 0):
            cp.start()

    nxt = jnp.minimum(b + 1, NB - 1)

    @pl.when(jnp.logical_and(b + 1 < NB, bexp_ref[2, nxt] != run))
    def _():
        for cp in _copies(bexp_ref[1, nxt], lax.rem(run + 1, 2)):
            cp.start()

    prv = jnp.maximum(b - 1, 0)
    first = jnp.logical_or(b == 0, bexp_ref[2, prv] != run)

    @pl.when(first)
    def _():
        for cp in _copies(bexp_ref[1, b], slot):
            cp.wait()

    x = xs_ref[...]
    gate = lax.dot_general(x, wg2[slot], (((1,), (0,)), ((), ())),
                           preferred_element_type=jnp.float32) + bg_ref[0]
    up = lax.dot_general(x, wu2[slot], (((1,), (0,)), ((), ())),
                         preferred_element_type=jnp.float32) + bu_ref[0]
    glu = jnp.minimum(gate, _LIMIT)
    lin = jnp.clip(up, -_LIMIT, _LIMIT)
    act = glu * jax.nn.sigmoid(_ALPHA * glu) * (lin + 1.0)
    ys_ref[...] = lax.dot_general(act, wd2[slot], (((1,), (0,)), ((), ())),
                                  preferred_element_type=jnp.float32) + bd_ref[0]


def _gmm(cntb, Xs, Wg, bg, Wu, bu, Wd, bd):
    grid_spec = pltpu.PrefetchScalarGridSpec(
        num_scalar_prefetch=1,
        grid=(NB,),
        in_specs=[
            pl.BlockSpec((BLK, D), lambda b, be: (b, 0)),
            pl.BlockSpec(memory_space=pl.ANY),
            pl.BlockSpec((1, 1, F), lambda b, be: (be[1, b], 0, 0)),
            pl.BlockSpec(memory_space=pl.ANY),
            pl.BlockSpec((1, 1, F), lambda b, be: (be[1, b], 0, 0)),
            pl.BlockSpec(memory_space=pl.ANY),
            pl.BlockSpec((1, 1, D), lambda b, be: (be[1, b], 0, 0)),
        ],
        out_specs=pl.BlockSpec((BLK, D), lambda b, be: (b, 0)),
        scratch_shapes=[
            pltpu.VMEM((2, D, F), jnp.float32),
            pltpu.VMEM((2, D, F), jnp.float32),
            pltpu.VMEM((2, F, D), jnp.float32),
            pltpu.SemaphoreType.DMA((2, 3)),
        ],
    )
    return pl.pallas_call(
        _gmm_body,
        grid_spec=grid_spec,
        out_shape=jax.ShapeDtypeStruct((SPAD, D), jnp.float32),
        compiler_params=pltpu.CompilerParams(
            dimension_semantics=("arbitrary",)),
    )(cntb, Xs, Wg, bg.reshape(E, 1, F), Wu, bu.reshape(E, 1, F),
      Wd, bd.reshape(E, 1, D))


def _comb_body(x_hbm, ys_hbm, meta, cntb, y_hbm,
               cnt_v, p0v, p1v, mv, xv, y0v, y1v, outv, sem):
    wid = lax.axis_index("s") * NC + lax.axis_index("c")
    base = wid * TPW
    rr = wid >> 1
    hh = wid & 1

    pltpu.sync_copy(cntb.at[0, pl.ds(0, 16)], cnt_v)
    pltpu.sync_copy(meta.at[pl.ds(rr, 1), pl.ds(0, E), pl.ds(0, TB)], mv)
    hh_f = jnp.broadcast_to(hh.astype(jnp.float32), (16,))
    _sc_positions(cnt_v, mv, hh_f, p0v, p1v)

    for c in range(TPW // 16):
        cp0 = pltpu.async_copy(ys_hbm.at[p0v.at[pl.ds(c * 16, 16)]], y0v, sem)
        cp1 = pltpu.async_copy(ys_hbm.at[p1v.at[pl.ds(c * 16, 16)]], y1v, sem)
        pltpu.sync_copy(x_hbm.at[pl.ds(base + c * 16, 16)], xv)
        cp0.wait()
        cp1.wait()
        a6 = mv[0, 6, pl.ds(c * 16, 16)]
        b6 = mv[0, 6, pl.ds(TPW + c * 16, 16)]
        a7 = mv[0, 7, pl.ds(c * 16, 16)]
        b7 = mv[0, 7, pl.ds(TPW + c * 16, 16)]
        w0g = (a6 + (b6 - a6) * hh_f) + (a7 + (b7 - a7) * hh_f)
        w1g = 1.0 - w0g
        for tl in range(16):
            w0s = _lane_splat(w0g, tl)
            w1s = _lane_splat(w1g, tl)

            def seg_body(f, _):
                sl = pl.ds(f * 16, 16)
                outv[tl, sl] = (xv[tl, sl] + w0s * y0v[tl, sl]
                                + w1s * y1v[tl, sl])
                return 0
            lax.fori_loop(0, D // 16, seg_body, 0)
        pltpu.sync_copy(outv, y_hbm.at[pl.ds(base + c * 16, 16)])


def _combine(x, Ys, meta, cntb):
    mesh = plsc.VectorSubcoreMesh(core_axis_name="c", subcore_axis_name="s")
    fn = functools.partial(
        pl.kernel, mesh=mesh,
        out_type=jax.ShapeDtypeStruct((T, D), jnp.float32),
        scratch_types=[
            pltpu.VMEM((16,), jnp.int32),
            pltpu.VMEM((TPW,), jnp.int32),
            pltpu.VMEM((TPW,), jnp.int32),
            pltpu.VMEM((1, E, TB), jnp.float32),
            pltpu.VMEM((16, D), jnp.float32),
            pltpu.VMEM((16, D), jnp.float32),
            pltpu.VMEM((16, D), jnp.float32),
            pltpu.VMEM((16, D), jnp.float32),
            pltpu.SemaphoreType.DMA,
        ],
    )(_comb_body)
    return fn(x, Ys, meta, cntb)


def kernel(x, norm_w, Wr, br, Wg, bg, Wu, bu, Wd, bd):
    t, meta, cntb = _routing(x, norm_w, Wr, br)
    Xs = _b2(t, meta, cntb)
    Ys = _gmm(cntb, Xs, Wg, bg, Wu, bu, Wd, bd)
    return _combine(x, Ys, meta, cntb)


# speedup vs baseline: 1.0544x; 1.0544x over previous
import functools

import jax
import jax.numpy as jnp
from jax import lax
from jax.experimental import pallas as pl
from jax.experimental.pallas import tpu as pltpu
from jax.experimental.pallas import tpu_sc as plsc

T = 2048
D = 1024
F = 1024
E = 8
BLK = 256
NB = 24
SPAD = NB * BLK
TB = 128
NTB = T // TB

_EPS = 1e-5
_ALPHA = 1.702
_LIMIT = 7.0

NC = 2
NS = 16
NW = NC * NS
TPW = T // NW
RPW = SPAD // NW
RCH = 48


def _routing_body(x_ref, nw_ref, wr_ref, br_ref,
                  t_ref, meta_ref, cnt_ref, carry_ref):
    b = pl.program_id(0)

    @pl.when(b == 0)
    def _():
        carry_ref[...] = jnp.zeros((1, TB), jnp.float32)

    x = x_ref[...]
    t = x * lax.rsqrt(jnp.mean(x * x, axis=1, keepdims=True) + _EPS) * nw_ref[...]
    t_ref[...] = t

    logits = lax.dot_general(t, wr_ref[...], (((1,), (1,)), ((), ())),
                             preferred_element_type=jnp.float32) + br_ref[...]
    iota_e = lax.broadcasted_iota(jnp.int32, (TB, E), 1).astype(jnp.float32)
    m0 = jnp.max(logits, axis=1, keepdims=True)
    e0 = jnp.min(jnp.where(logits >= m0, iota_e, 1e9), axis=1, keepdims=True)
    masked = jnp.where(iota_e == e0, -1e30, logits)
    m1 = jnp.max(masked, axis=1, keepdims=True)
    e1 = jnp.min(jnp.where(masked >= m1, iota_e, 1e9), axis=1, keepdims=True)
    w0 = jax.nn.sigmoid(m0 - m1)

    lane = lax.broadcasted_iota(jnp.int32, (TB, TB), 1).astype(jnp.float32)
    oh0 = (lane == e0).astype(jnp.float32)
    oh1 = (lane == e1).astype(jnp.float32)
    m_mat = oh0 + oh1
    row_i = lax.broadcasted_iota(jnp.int32, (TB, TB), 0).astype(jnp.float32)
    tri = (lane < row_i).astype(jnp.float32)
    carry = carry_ref[...]
    rank = lax.dot_general(tri, m_mat, (((1,), (0,)), ((), ())),
                           preferred_element_type=jnp.float32) + carry
    p0 = jnp.sum(rank * oh0, axis=1, keepdims=True)
    p1 = jnp.sum(rank * oh1, axis=1, keepdims=True)
    new_carry = carry + jnp.sum(m_mat, axis=0, keepdims=True)
    carry_ref[...] = new_carry

    p0hi = jnp.floor(p0 * (1.0 / 128.0))
    p0lo = p0 - 128.0 * p0hi
    p1hi = jnp.floor(p1 * (1.0 / 128.0))
    p1lo = p1 - 128.0 * p1hi
    w0hi = jnp.floor(w0 * 256.0 + 0.5) * (1.0 / 256.0)
    w0lo = w0 - w0hi
    cols = jnp.concatenate(
        [e0, e1, p0lo, p1lo, p0hi, p1hi, w0hi, w0lo], axis=1)
    eye = (lane == row_i).astype(jnp.float32)
    meta = lax.dot_general(cols, eye, (((0,), (0,)), ((), ())),
                           preferred_element_type=jnp.float32)
    meta_ref[...] = meta.reshape(1, E, TB)

    nblk = jnp.floor((new_carry + (BLK - 1)) * (1.0 / BLK))
    lte = (row_i <= lane).astype(jnp.float32)
    end_row = lax.dot_general(nblk, lte, (((1,), (0,)), ((), ())),
                              preferred_element_type=jnp.float32)
    end_col = lax.dot_general(eye, end_row, (((1,), (1,)), ((), ())),
                              preferred_element_type=jnp.float32)
    ge = (lane >= end_col).astype(jnp.float32) * (row_i < E).astype(jnp.float32)
    ones_row = jnp.ones((1, TB), jnp.float32)
    bexp_row = lax.dot_general(ones_row, ge, (((1,), (0,)), ((), ())),
                               preferred_element_type=jnp.float32)
    bexp_row = jnp.minimum(bexp_row, float(E - 1))
    shr = (row_i == lane - 1.0).astype(jnp.float32)
    prev_row = lax.dot_general(bexp_row, shr, (((1,), (0,)), ((), ())),
                               preferred_element_type=jnp.float32)
    lr = lane[0:1, :]
    change_row = ((bexp_row != prev_row).astype(jnp.float32)
                  * (lr > 0.0).astype(jnp.float32))
    runidx_row = lax.dot_general(change_row, lte, (((1,), (0,)), ((), ())),
                                 preferred_element_type=jnp.float32)
    nz_row = (new_carry > 0.0).astype(jnp.float32)
    ltr = (row_i < lane).astype(jnp.float32)
    rank_row = lax.dot_general(nz_row, ltr, (((1,), (0,)), ((), ())),
                               preferred_element_type=jnp.float32)
    rank_col = lax.dot_general(eye, rank_row, (((1,), (1,)), ((), ())),
                               preferred_element_type=jnp.float32)
    nz_col = lax.dot_general(eye, nz_row, (((1,), (1,)), ((), ())),
                             preferred_element_type=jnp.float32)
    ind = (lane == rank_col).astype(jnp.float32) * nz_col
    runexp_row = lax.dot_general(lr, ind, (((1,), (0,)), ((), ())),
                                 preferred_element_type=jnp.float32)
    ones_mat = jnp.ones((TB, TB), jnp.float32)
    nruns_row = lax.dot_general(nz_row, ones_mat, (((1,), (0,)), ((), ())),
                                preferred_element_type=jnp.float32)
    cnt_ref[...] = jnp.concatenate(
        [new_carry, bexp_row, runidx_row, runexp_row, nruns_row,
         jnp.zeros((E - 5, TB), jnp.float32)],
        axis=0).astype(jnp.int32)


def _routing(x, norm_w, Wr, br):
    out_shape = [
        jax.ShapeDtypeStruct((T, D), jnp.float32),
        jax.ShapeDtypeStruct((NTB, E, TB), jnp.float32),
        jax.ShapeDtypeStruct((E, TB), jnp.int32),
    ]
    return pl.pallas_call(
        _routing_body,
        grid=(NTB,),
        in_specs=[
            pl.BlockSpec((TB, D), lambda b: (b, 0)),
            pl.BlockSpec((1, D), lambda b: (0, 0)),
            pl.BlockSpec((E, D), lambda b: (0, 0)),
            pl.BlockSpec((1, E), lambda b: (0, 0)),
        ],
        out_specs=[
            pl.BlockSpec((TB, D), lambda b: (b, 0)),
            pl.BlockSpec((1, E, TB), lambda b: (b, 0, 0)),
            pl.BlockSpec((E, TB), lambda b: (0, 0)),
        ],
        out_shape=out_shape,
        scratch_shapes=[pltpu.VMEM((1, TB), jnp.float32)],
        compiler_params=pltpu.CompilerParams(
            dimension_semantics=("arbitrary",)),
    )(x, norm_w.reshape(1, D), Wr, br.reshape(1, E))


def _vgather(vec, idx):
    dnums = lax.GatherDimensionNumbers(
        offset_dims=(), collapsed_slice_dims=(0,), start_index_map=(0,))
    return lax.gather(vec, idx.reshape(16, 1), dnums, (1,),
                      mode=lax.GatherScatterMode.PROMISE_IN_BOUNDS)


def _lane_splat(vec, lane):
    return _vgather(vec, jnp.full((16,), lane, jnp.int32))


def _cumsum16(vec):
    iota16 = lax.broadcasted_iota(jnp.int32, (16,), 0)
    x = vec
    for k in (1, 2, 4, 8):
        sh = _vgather(x, jnp.maximum(iota16 - k, 0))
        x = x + jnp.where(iota16 >= k, sh, jnp.zeros_like(x))
    return x


def _sc_positions(cnt_v, mv, hh_f, pb0, pb1):
    iota16 = lax.broadcasted_iota(jnp.int32, (16,), 0)
    zero16 = jnp.zeros((16,), jnp.int32)
    c_i = jnp.where(iota16 < E, cnt_v[...], zero16)
    shift = BLK.bit_length() - 1
    padded = ((c_i + (BLK - 1)) >> shift) << shift
    incl = _cumsum16(padded)
    off = incl - padded
    offs = [_lane_splat(off, e) for e in range(E)]

    def _half(row, g):
        a = mv[0, row, pl.ds(16 * g, 16)]
        b = mv[0, row, pl.ds(TPW + 16 * g, 16)]
        return a + (b - a) * hh_f

    for g in range(TPW // 16):
        e0g = _half(0, g).astype(jnp.int32)
        e1g = _half(1, g).astype(jnp.int32)
        acc0 = (_half(2, g) + 128.0 * _half(4, g)).astype(jnp.int32)
        acc1 = (_half(3, g) + 128.0 * _half(5, g)).astype(jnp.int32)
        for e in range(E):
            acc0 = acc0 + jnp.where(e0g == e, offs[e], zero16)
            acc1 = acc1 + jnp.where(e1g == e, offs[e], zero16)
        pb0[pl.ds(16 * g, 16)] = acc0
        pb1[pl.ds(16 * g, 16)] = acc1


def _b2_body(t_hbm, meta, cntb, xs_hbm, cnt_v, mv, pb0, pb1, rows, sem):
    wid = lax.axis_index("s") * NC + lax.axis_index("c")
    base = wid * TPW
    rr = wid >> 1
    hh = wid & 1

    pltpu.sync_copy(cntb.at[0, pl.ds(0, 16)], cnt_v)
    pltpu.sync_copy(meta.at[pl.ds(rr, 1), pl.ds(0, E), pl.ds(0, TB)], mv)
    cptr = pltpu.async_copy(t_hbm.at[pl.ds(base, TPW)], rows, sem)
    hh_f = jnp.broadcast_to(hh.astype(jnp.float32), (16,))
    _sc_positions(cnt_v, mv, hh_f, pb0, pb1)
    cptr.wait()
    cp0 = pltpu.async_copy(rows, xs_hbm.at[pb0], sem)
    cp1 = pltpu.async_copy(rows, xs_hbm.at[pb1], sem)
    cp0.wait()
    cp1.wait()


def _b2(t, meta, cntb):
    mesh = plsc.VectorSubcoreMesh(core_axis_name="c", subcore_axis_name="s")
    fn = functools.partial(
        pl.kernel, mesh=mesh,
        out_type=jax.ShapeDtypeStruct((SPAD, D), jnp.float32),
        scratch_types=[
            pltpu.VMEM((16,), jnp.int32),
            pltpu.VMEM((1, E, TB), jnp.float32),
            pltpu.VMEM((TPW,), jnp.int32),
            pltpu.VMEM((TPW,), jnp.int32),
            pltpu.VMEM((TPW, D), jnp.float32),
            pltpu.SemaphoreType.DMA,
        ],
    )(_b2_body)
    return fn(t, meta, cntb)


def _gmm_body(bexp_ref, xs_ref, wg_hbm, bg_ref, wu_hbm, bu_ref,
              wd_hbm, bd_ref, ys_ref, wg2, wu2, wd2, sems):
    b = pl.program_id(0)
    run = bexp_ref[2, b]
    slot = lax.rem(run, 2)

    def _copies(e, s):
        return (pltpu.make_async_copy(wg_hbm.at[e], wg2.at[s], sems.at[s, 0]),
                pltpu.make_async_copy(wu_hbm.at[e], wu2.at[s], sems.at[s, 1]),
                pltpu.make_async_copy(wd_hbm.at[e], wd2.at[s], sems.at[s, 2]))

    @pl.when(b == 0)
    def _():
        for cp in _copies(bexp_ref[1, 0], 0):
            cp.start()

    prv = jnp.maximum(b - 1, 0)
    first = jnp.logical_or(b == 0, bexp_ref[2, prv] != run)
    nxt_run = jnp.minimum(run + 1, E - 1)

    @pl.when(jnp.logical_and(first, run + 1 < bexp_ref[4, 0]))
    def _():
        for cp in _copies(bexp_ref[3, nxt_run], lax.rem(run + 1, 2)):
            cp.start()

    @pl.when(first)
    def _():
        for cp in _copies(bexp_ref[1, b], slot):
            cp.wait()

    x = xs_ref[...]
    gate = lax.dot_general(x, wg2[slot], (((1,), (0,)), ((), ())),
                           preferred_element_type=jnp.float32) + bg_ref[0]
    up = lax.dot_general(x, wu2[slot], (((1,), (0,)), ((), ())),
                         preferred_element_type=jnp.float32) + bu_ref[0]
    glu = jnp.minimum(gate, _LIMIT)
    lin = jnp.clip(up, -_LIMIT, _LIMIT)
    act = glu * jax.nn.sigmoid(_ALPHA * glu) * (lin + 1.0)
    ys_ref[...] = lax.dot_general(act, wd2[slot], (((1,), (0,)), ((), ())),
                                  preferred_element_type=jnp.float32) + bd_ref[0]


def _gmm(cntb, Xs, Wg, bg, Wu, bu, Wd, bd):
    grid_spec = pltpu.PrefetchScalarGridSpec(
        num_scalar_prefetch=1,
        grid=(NB,),
        in_specs=[
            pl.BlockSpec((BLK, D), lambda b, be: (b, 0)),
            pl.BlockSpec(memory_space=pl.ANY),
            pl.BlockSpec((1, 1, F), lambda b, be: (be[1, b], 0, 0)),
            pl.BlockSpec(memory_space=pl.ANY),
            pl.BlockSpec((1, 1, F), lambda b, be: (be[1, b], 0, 0)),
            pl.BlockSpec(memory_space=pl.ANY),
            pl.BlockSpec((1, 1, D), lambda b, be: (be[1, b], 0, 0)),
        ],
        out_specs=pl.BlockSpec((BLK, D), lambda b, be: (b, 0)),
        scratch_shapes=[
            pltpu.VMEM((2, D, F), jnp.float32),
            pltpu.VMEM((2, D, F), jnp.float32),
            pltpu.VMEM((2, F, D), jnp.float32),
            pltpu.SemaphoreType.DMA((2, 3)),
        ],
    )
    return pl.pallas_call(
        _gmm_body,
        grid_spec=grid_spec,
        out_shape=jax.ShapeDtypeStruct((SPAD, D), jnp.float32),
        compiler_params=pltpu.CompilerParams(
            dimension_semantics=("arbitrary",)),
    )(cntb, Xs, Wg, bg.reshape(E, 1, F), Wu, bu.reshape(E, 1, F),
      Wd, bd.reshape(E, 1, D))


def _comb_body(x_hbm, ys_hbm, meta, cntb, y_hbm,
               cnt_v, p0v, p1v, mv, xv, y0v, y1v, outv, sem):
    wid = lax.axis_index("s") * NC + lax.axis_index("c")
    base = wid * TPW
    rr = wid >> 1
    hh = wid & 1

    pltpu.sync_copy(cntb.at[0, pl.ds(0, 16)], cnt_v)
    pltpu.sync_copy(meta.at[pl.ds(rr, 1), pl.ds(0, E), pl.ds(0, TB)], mv)
    hh_f = jnp.broadcast_to(hh.astype(jnp.float32), (16,))
    _sc_positions(cnt_v, mv, hh_f, p0v, p1v)

    for c in range(TPW // 16):
        cp0 = pltpu.async_copy(ys_hbm.at[p0v.at[pl.ds(c * 16, 16)]], y0v, sem)
        cp1 = pltpu.async_copy(ys_hbm.at[p1v.at[pl.ds(c * 16, 16)]], y1v, sem)
        pltpu.sync_copy(x_hbm.at[pl.ds(base + c * 16, 16)], xv)
        cp0.wait()
        cp1.wait()
        a6 = mv[0, 6, pl.ds(c * 16, 16)]
        b6 = mv[0, 6, pl.ds(TPW + c * 16, 16)]
        a7 = mv[0, 7, pl.ds(c * 16, 16)]
        b7 = mv[0, 7, pl.ds(TPW + c * 16, 16)]
        w0g = (a6 + (b6 - a6) * hh_f) + (a7 + (b7 - a7) * hh_f)
        w1g = 1.0 - w0g
        for tl in range(16):
            w0s = _lane_splat(w0g, tl)
            w1s = _lane_splat(w1g, tl)

            def seg_body(f, _):
                sl = pl.ds(f * 16, 16)
                outv[tl, sl] = (xv[tl, sl] + w0s * y0v[tl, sl]
                                + w1s * y1v[tl, sl])
                return 0
            lax.fori_loop(0, D // 16, seg_body, 0)
        pltpu.sync_copy(outv, y_hbm.at[pl.ds(base + c * 16, 16)])


def _combine(x, Ys, meta, cntb):
    mesh = plsc.VectorSubcoreMesh(core_axis_name="c", subcore_axis_name="s")
    fn = functools.partial(
        pl.kernel, mesh=mesh,
        out_type=jax.ShapeDtypeStruct((T, D), jnp.float32),
        scratch_types=[
            pltpu.VMEM((16,), jnp.int32),
            pltpu.VMEM((TPW,), jnp.int32),
            pltpu.VMEM((TPW,), jnp.int32),
            pltpu.VMEM((1, E, TB), jnp.float32),
            pltpu.VMEM((16, D), jnp.float32),
            pltpu.VMEM((16, D), jnp.float32),
            pltpu.VMEM((16, D), jnp.float32),
            pltpu.VMEM((16, D), jnp.float32),
            pltpu.SemaphoreType.DMA,
        ],
    )(_comb_body)
    return fn(x, Ys, meta, cntb)


def kernel(x, norm_w, Wr, br, Wg, bg, Wu, bu, Wd, bd):
    t, meta, cntb = _routing(x, norm_w, Wr, br)
    Xs = _b2(t, meta, cntb)
    Ys = _gmm(cntb, Xs, Wg, bg, Wu, bu, Wd, bd)
    return _combine(x, Ys, meta, cntb)
